# baseline (speedup 1.0000x reference)
import functools

import jax
import jax.numpy as jnp
from jax import lax
from jax.experimental import pallas as pl
from jax.experimental.pallas import tpu as pltpu
from jax.experimental.pallas import tpu_sc as plsc

N = 4096
K = 8192
BN = 256
NB = N // BN

NC = 2
NS = 16
NW = NC * NS
QPW = N // NW
L = 16


def _argmin_body(q_ref, r_ref, idx_ref, rt_s):
    nb = pl.program_id(0)

    @pl.when(nb == 0)
    def _build_rt():
        rt_s[...] = jnp.transpose(r_ref[...])

    qx = q_ref[:, 0:1]
    qy = q_ref[:, 1:2]
    rx = rt_s[0:1, :]
    ry = rt_s[1:2, :]
    dx = rx - qx
    dy = ry - qy
    d2 = dx * dx + dy * dy

    m = jnp.min(d2, axis=1, keepdims=True)
    fiota = jax.lax.broadcasted_iota(jnp.int32, (BN, K), 1).astype(jnp.float32)
    fidx = jnp.min(jnp.where(d2 <= m, fiota, float(K)), axis=1, keepdims=True)
    row = jnp.transpose(fidx.astype(jnp.int32))
    idx_ref[...] = row.reshape(1, 1, BN)


def _sc_gather_body(idx_hbm, dall_hbm,
                    d2_out, b_out,
                    idx_v, dp_v, b_v, nw_v, d2_v, gb_v, s0, s1, s2, s3):
    wid = lax.axis_index("s") * NC + lax.axis_index("c")
    base = wid * QPW
    c0 = pltpu.async_copy(
        idx_hbm.at[wid // (BN // QPW), 0,
                   pl.ds((wid % (BN // QPW)) * QPW, QPW)], idx_v, s0)
    c1 = pltpu.async_copy(dall_hbm.at[pl.ds(0, 2 * K)], dp_v, s1)
    c2 = pltpu.async_copy(dall_hbm.at[pl.ds(2 * K, K)], b_v, s2)
    c3 = pltpu.async_copy(
        dall_hbm.at[pl.ds(3 * K + 2 * base, 2 * QPW)], nw_v, s3)
    c0.wait()
    c1.wait()
    c2.wait()
    c3.wait()
    li = lax.iota(jnp.int32, L)
    for j in range(QPW // L):
        sl = pl.ds(j * L, L)
        iv = idx_v[sl]
        iv2 = iv * 2
        gx = plsc.load_gather(dp_v, [iv2])
        gy = plsc.load_gather(dp_v, [iv2 + 1])
        gb = plsc.load_gather(b_v, [iv])
        nl = li * 2 + (2 * L) * j
        nx = plsc.load_gather(nw_v, [nl])
        ny = plsc.load_gather(nw_v, [nl + 1])
        dx = nx - gx
        dy = ny - gy
        d2_v[sl] = dx * dx + dy * dy
        gb_v[sl] = gb
    pltpu.sync_copy(d2_v, d2_out.at[wid])
    pltpu.sync_copy(gb_v, b_out.at[wid])


def _reduce_body(d2_ref, b_ref, out_ref):
    pp = jnp.sqrt(d2_ref[...])
    b = b_ref[...]
    out_ref[0, 0] = jnp.sum(pp * b) / jnp.sum(b)


@jax.jit
def _track_loss(q, dict_ref, dall):
    idx = pl.pallas_call(
        _argmin_body,
        grid=(NB,),
        in_specs=[
            pl.BlockSpec((BN, 2), lambda nb: (nb, 0)),
            pl.BlockSpec((K, 2), lambda nb: (0, 0)),
        ],
        out_specs=pl.BlockSpec((1, 1, BN), lambda nb: (nb, 0, 0)),
        out_shape=jax.ShapeDtypeStruct((NB, 1, BN), jnp.int32),
        scratch_shapes=[
            pltpu.VMEM((2, K), jnp.float32),
        ],
        compiler_params=pltpu.CompilerParams(
            dimension_semantics=("arbitrary",),
        ),
    )(q, dict_ref)

    sc_gather = functools.partial(
        pl.kernel,
        out_type=(
            jax.ShapeDtypeStruct((NW, QPW), jnp.float32),
            jax.ShapeDtypeStruct((NW, QPW), jnp.float32),
        ),
        mesh=plsc.VectorSubcoreMesh(core_axis_name="c", subcore_axis_name="s"),
        compiler_params=pltpu.CompilerParams(needs_layout_passes=False),
        scratch_types=[
            pltpu.VMEM((QPW,), jnp.int32),
            pltpu.VMEM((2 * K,), jnp.float32),
            pltpu.VMEM((K,), jnp.float32),
            pltpu.VMEM((2 * QPW,), jnp.float32),
            pltpu.VMEM((QPW,), jnp.float32),
            pltpu.VMEM((QPW,), jnp.float32),
            pltpu.SemaphoreType.DMA,
            pltpu.SemaphoreType.DMA,
            pltpu.SemaphoreType.DMA,
            pltpu.SemaphoreType.DMA,
        ],
    )(_sc_gather_body)
    d2g, bg = sc_gather(idx, dall)

    out = pl.pallas_call(
        _reduce_body,
        out_specs=pl.BlockSpec(memory_space=pltpu.SMEM),
        out_shape=jax.ShapeDtypeStruct((1, 1), jnp.float32),
    )(d2g, bg)
    return out[0, 0]


def kernel(flat_origin_curves, flat_new_curves, dict_points, dict_ref, dict_bool):
    dall = jnp.concatenate([
        dict_points.reshape(2 * K),
        dict_bool.astype(jnp.float32),
        flat_new_curves.reshape(2 * N),
    ])
    return _track_loss(flat_origin_curves, dict_ref, dall)


# speedup vs baseline: 1.0966x; 1.0966x over previous
import functools

import jax
import jax.numpy as jnp
from jax import lax
from jax.experimental import pallas as pl
from jax.experimental.pallas import tpu as pltpu
from jax.experimental.pallas import tpu_sc as plsc

N = 4096
K = 8192
BN = 256
NB = N // BN

NC = 2
NS = 16
NW = NC * NS
QPW = N // NW
L = 16


def _argmin_body(q_ref, rt_ref, idx_ref):
    qx = q_ref[:, 0:1]
    qy = q_ref[:, 1:2]
    rx = rt_ref[0:1, :]
    ry = rt_ref[1:2, :]
    dx = rx - qx
    dy = ry - qy
    d2 = dx * dx + dy * dy

    m = jnp.min(d2, axis=1, keepdims=True)
    fiota = jax.lax.broadcasted_iota(jnp.int32, (BN, K), 1).astype(jnp.float32)
    fidx = jnp.min(jnp.where(d2 <= m, fiota, float(K)), axis=1, keepdims=True)
    idx_ref[...] = fidx.astype(jnp.int32)


def _sc_gather_body(idx_hbm, dall_hbm,
                    d2_out, b_out,
                    idx_v, dp_v, b_v, nw_v, d2_v, gb_v, s0, s1, s2, s3):
    wid = lax.axis_index("s") * NC + lax.axis_index("c")
    base = wid * QPW
    c0 = pltpu.async_copy(idx_hbm.at[pl.ds(base, QPW)], idx_v, s0)
    c1 = pltpu.async_copy(dall_hbm.at[pl.ds(0, 2 * K)], dp_v, s1)
    c2 = pltpu.async_copy(dall_hbm.at[pl.ds(2 * K, K)], b_v, s2)
    c3 = pltpu.async_copy(
        dall_hbm.at[pl.ds(3 * K + 2 * base, 2 * QPW)], nw_v, s3)
    c0.wait()
    c1.wait()
    c2.wait()
    c3.wait()
    li = lax.iota(jnp.int32, L)
    for j in range(QPW // L):
        sl = pl.ds(j * L, L)
        iv = idx_v[sl]
        iv2 = iv * 2
        gx = plsc.load_gather(dp_v, [iv2])
        gy = plsc.load_gather(dp_v, [iv2 + 1])
        gb = plsc.load_gather(b_v, [iv])
        nl = li * 2 + (2 * L) * j
        nx = plsc.load_gather(nw_v, [nl])
        ny = plsc.load_gather(nw_v, [nl + 1])
        dx = nx - gx
        dy = ny - gy
        d2_v[sl] = dx * dx + dy * dy
        gb_v[sl] = gb
    pltpu.sync_copy(d2_v, d2_out.at[wid])
    pltpu.sync_copy(gb_v, b_out.at[wid])


def _reduce_body(d2_ref, b_ref, out_ref):
    pp = jnp.sqrt(d2_ref[...])
    b = b_ref[...]
    out_ref[0, 0] = jnp.sum(pp * b) / jnp.sum(b)


@jax.jit
def _track_loss(q, rt, dall):
    idx2d = pl.pallas_call(
        _argmin_body,
        grid=(NB,),
        in_specs=[
            pl.BlockSpec((BN, 2), lambda nb: (nb, 0)),
            pl.BlockSpec((2, K), lambda nb: (0, 0)),
        ],
        out_specs=pl.BlockSpec((BN, 1), lambda nb: (nb, 0)),
        out_shape=jax.ShapeDtypeStruct((N, 1), jnp.int32),
        compiler_params=pltpu.CompilerParams(
            dimension_semantics=("arbitrary",),
        ),
    )(q, rt)
    idx = idx2d.reshape(N)

    sc_gather = functools.partial(
        pl.kernel,
        out_type=(
            jax.ShapeDtypeStruct((NW, QPW), jnp.float32),
            jax.ShapeDtypeStruct((NW, QPW), jnp.float32),
        ),
        mesh=plsc.VectorSubcoreMesh(core_axis_name="c", subcore_axis_name="s"),
        compiler_params=pltpu.CompilerParams(needs_layout_passes=False),
        scratch_types=[
            pltpu.VMEM((QPW,), jnp.int32),
            pltpu.VMEM((2 * K,), jnp.float32),
            pltpu.VMEM((K,), jnp.float32),
            pltpu.VMEM((2 * QPW,), jnp.float32),
            pltpu.VMEM((QPW,), jnp.float32),
            pltpu.VMEM((QPW,), jnp.float32),
            pltpu.SemaphoreType.DMA,
            pltpu.SemaphoreType.DMA,
            pltpu.SemaphoreType.DMA,
            pltpu.SemaphoreType.DMA,
        ],
    )(_sc_gather_body)
    d2g, bg = sc_gather(idx, dall)

    out = pl.pallas_call(
        _reduce_body,
        out_specs=pl.BlockSpec(memory_space=pltpu.SMEM),
        out_shape=jax.ShapeDtypeStruct((1, 1), jnp.float32),
    )(d2g, bg)
    return out[0, 0]


def kernel(flat_origin_curves, flat_new_curves, dict_points, dict_ref, dict_bool):
    dall = jnp.concatenate([
        dict_points.reshape(2 * K),
        dict_bool.astype(jnp.float32),
        flat_new_curves.reshape(2 * N),
    ])
    return _track_loss(flat_origin_curves, dict_ref.T, dall)
